# baseline (speedup 1.0000x reference)
import functools

import jax
import jax.numpy as jnp
import numpy as np
from jax.experimental import pallas as pl
from jax.experimental.pallas import tpu as pltpu


def _round_up(v, m):
    return ((v + m - 1) // m) * m


def _tree_reduce(vals, op):
    vals = list(vals)
    while len(vals) > 1:
        nxt = [op(vals[i], vals[i + 1]) for i in range(0, len(vals) - 1, 2)]
        if len(vals) % 2:
            nxt.append(vals[-1])
        vals = nxt
    return vals[0]


def _conv_mat_tables(wth, kk):
    p = kk // 2
    rows, cols = 4 * wth, 2 * wth
    dd = np.arange(5)[:, None, None]
    rr = np.arange(rows)[None, :, None]
    cc = np.arange(cols)[None, None, :]
    pi = rr // (2 * wth)
    pp = (rr % (2 * wth)) // wth
    q = cc // wth
    j = cc % wth
    d = rr % wth - j
    ki = 2 * (dd - 2) + pp - q + p
    kj = d + p
    valid = (kj >= 0) & (kj < kk) & (ki >= 0) & (ki < kk)
    idx = (pi * kk * kk + np.clip(ki, 0, kk - 1) * kk
           + np.clip(kj, 0, kk - 1))
    idx = np.broadcast_to(idx, (5, rows, cols)).astype(np.int32)
    mask = np.broadcast_to(valid, (5, rows, cols)).astype(np.float32)
    return idx, mask


def _build_conv_mats(w0, w1c, wth, kk):
    idx, mask = _conv_mat_tables(wth, kk)
    w_all = jnp.concatenate([w0.reshape(-1), w1c.reshape(-1)])
    return jnp.take(w_all, jnp.asarray(idx), axis=0) * jnp.asarray(mask)


def _sa_body(x_ref, m_ref, corr_ref, s_ref, o_ref, pad_ref, *,
             C, HPAIR, W2):
    rows = 8
    cpi = min(4, C // rows)
    n_iter = C // (rows * cpi)
    step_c = rows * cpi

    for s in range(HPAIR):
        lane0 = s * W2

        def body(i, carry, _lane0=lane0):
            m, su = carry
            base = pl.multiple_of(i * step_c, step_c)
            cs = [x_ref[0, pl.ds(base + k * rows, rows), pl.ds(_lane0, W2)]
                  for k in range(cpi)]
            m1 = _tree_reduce(cs, jnp.maximum)
            s1 = _tree_reduce(cs, jnp.add)
            return jnp.maximum(m, m1), su + s1

        init = (jnp.full((rows, W2), -jnp.inf, jnp.float32),
                jnp.zeros((rows, W2), jnp.float32))
        mx, sm = jax.lax.fori_loop(0, n_iter, body, init, unroll=2)
        pad_ref[pl.ds(2 + s, 1), 0:W2] = jnp.max(mx, axis=0, keepdims=True)
        pad_ref[pl.ds(2 + s, 1), W2:2 * W2] = jnp.sum(sm, axis=0,
                                                      keepdims=True)

    blk = pad_ref[pl.ds(2, HPAIR), :]
    c0 = jnp.mean(blk[:, 0:W2])
    c1 = jnp.mean(blk[:, W2:2 * W2])
    lane = jax.lax.broadcasted_iota(jnp.int32, (HPAIR, 2 * W2), 1)
    offs = jnp.where(lane < W2, c0, c1)
    pad_ref[0:2, :] = jnp.zeros((2, 2 * W2), jnp.float32)
    pad_ref[pl.ds(2 + HPAIR, 2), :] = jnp.zeros((2, 2 * W2), jnp.float32)
    pad_ref[pl.ds(2, HPAIR), :] = blk - offs

    acc = None
    for dlt in range(5):
        win = pad_ref[pl.ds(dlt, HPAIR), :]
        mm = jnp.dot(win, m_ref[dlt], preferred_element_type=jnp.float32)
        acc = mm if acc is None else acc + mm

    z = acc + c0 * corr_ref[0] + c1 * corr_ref[1] + s_ref[0]
    o_ref[0, 0] = jax.nn.sigmoid(z).astype(o_ref.dtype)


def _spatial_attention(x, weight, bias):
    B, C, H, W = x.shape
    kk = weight.shape[2]
    p = kk // 2
    assert H % 2 == 0 and W == 64 and C % 8 == 0
    hpair = H // 2
    w2 = 2 * W

    x_flat = x.reshape(B, C, H * W)

    w0 = weight[0, 0].astype(jnp.float32)
    w1c = weight[0, 1].astype(jnp.float32) * (1.0 / C)
    mats = _build_conv_mats(w0, w1c, W, kk)

    hh = np.arange(H)[:, None] + np.arange(kk)[None, :] - p
    um = ((hh >= 0) & (hh < H)).astype(np.float32)
    wwv = np.arange(W)[:, None] + np.arange(kk)[None, :] - p
    vm = ((wwv >= 0) & (wwv < W)).astype(np.float32)
    s0 = um @ w0 @ vm.T
    s1 = um @ w1c @ vm.T
    corr = jnp.stack([s0.reshape(hpair, w2), s1.reshape(hpair, w2)])

    bias_s = bias.reshape(-1).astype(jnp.float32)

    pad_rows = _round_up(hpair + 4, 8)
    body = functools.partial(_sa_body, C=C, HPAIR=hpair, W2=w2)

    cost = pl.CostEstimate(
        flops=int(B * H * W * (2 * C + 4 * kk * kk + 4)),
        transcendentals=int(B * H * W),
        bytes_accessed=int(B * (C + 1) * H * W * 4 + mats.size * 4),
    )

    out = pl.pallas_call(
        body,
        out_shape=jax.ShapeDtypeStruct((B, 1, hpair, w2), x.dtype),
        grid=(B,),
        in_specs=[
            pl.BlockSpec((1, C, H * W), lambda b: (b, 0, 0)),
            pl.BlockSpec((5, 4 * W, w2), lambda b: (0, 0, 0)),
            pl.BlockSpec((2, hpair, w2), lambda b: (0, 0, 0)),
            pl.BlockSpec(memory_space=pltpu.MemorySpace.SMEM),
        ],
        out_specs=pl.BlockSpec((1, 1, hpair, w2), lambda b: (b, 0, 0, 0)),
        scratch_shapes=[
            pltpu.VMEM((pad_rows, 2 * w2), jnp.float32),
        ],
        compiler_params=pltpu.CompilerParams(
            dimension_semantics=("parallel",),
            vmem_limit_bytes=32 * 1024 * 1024),
        cost_estimate=cost,
    )(x_flat, mats, corr, bias_s)

    return out.reshape(B, 1, H, W)


def kernel(x, weight, bias):
    return _spatial_attention(x, weight, bias)


# speedup vs baseline: 7.6558x; 7.6558x over previous
import functools

import jax
import jax.numpy as jnp
import numpy as np
from jax.experimental import pallas as pl
from jax.experimental.pallas import tpu as pltpu


def _round_up(v, m):
    return ((v + m - 1) // m) * m


def _tree_reduce(vals, op):
    vals = list(vals)
    while len(vals) > 1:
        nxt = [op(vals[i], vals[i + 1]) for i in range(0, len(vals) - 1, 2)]
        if len(vals) % 2:
            nxt.append(vals[-1])
        vals = nxt
    return vals[0]


def _build_conv_mats(w_all, wth, kk):
    p = kk // 2
    a = np.zeros((5, 2, 2, kk), np.float32)
    for dlt in range(5):
        for pp in (0, 1):
            for q in (0, 1):
                ki = 2 * (dlt - 2) + pp - q + p
                if 0 <= ki < kk:
                    a[dlt, pp, q, ki] = 1.0
    jd = np.arange(wth)[:, None]
    jj = np.arange(wth)[None, :]
    tb = np.stack([(jd - jj + p == kj).astype(np.float32)
                   for kj in range(kk)])
    t1 = jnp.einsum('dpqk,ikl->dpqil', a, w_all)
    mats = jnp.einsum('dpqil,ljm->dipjqm', t1, tb)
    return mats.reshape(5, 4 * wth, 2 * wth)


def _sa_body(x_ref, m_ref, corr_ref, s_ref, o_ref, pad_ref, *,
             C, HPAIR, W2):
    rows = 8
    cpi = min(4, C // rows)
    n_iter = C // (rows * cpi)
    step_c = rows * cpi

    for s in range(HPAIR):
        lane0 = s * W2

        def body(i, carry, _lane0=lane0):
            m, su = carry
            base = pl.multiple_of(i * step_c, step_c)
            cs = [x_ref[0, pl.ds(base + k * rows, rows), pl.ds(_lane0, W2)]
                  for k in range(cpi)]
            m1 = _tree_reduce(cs, jnp.maximum)
            s1 = _tree_reduce(cs, jnp.add)
            return jnp.maximum(m, m1), su + s1

        init = (jnp.full((rows, W2), -jnp.inf, jnp.float32),
                jnp.zeros((rows, W2), jnp.float32))
        mx, sm = jax.lax.fori_loop(0, n_iter, body, init, unroll=2)
        pad_ref[pl.ds(2 + s, 1), 0:W2] = jnp.max(mx, axis=0, keepdims=True)
        pad_ref[pl.ds(2 + s, 1), W2:2 * W2] = jnp.sum(sm, axis=0,
                                                      keepdims=True)

    blk = pad_ref[pl.ds(2, HPAIR), :]
    c0 = jnp.mean(blk[:, 0:W2])
    c1 = jnp.mean(blk[:, W2:2 * W2])
    lane = jax.lax.broadcasted_iota(jnp.int32, (HPAIR, 2 * W2), 1)
    offs = jnp.where(lane < W2, c0, c1)
    pad_ref[0:2, :] = jnp.zeros((2, 2 * W2), jnp.float32)
    pad_ref[pl.ds(2 + HPAIR, 2), :] = jnp.zeros((2, 2 * W2), jnp.float32)
    pad_ref[pl.ds(2, HPAIR), :] = blk - offs

    acc = None
    for dlt in range(5):
        win = pad_ref[pl.ds(dlt, HPAIR), :]
        mm = jnp.dot(win, m_ref[dlt], preferred_element_type=jnp.float32)
        acc = mm if acc is None else acc + mm

    z = acc + c0 * corr_ref[0] + c1 * corr_ref[1] + s_ref[0]
    o_ref[0, 0] = jax.nn.sigmoid(z).astype(o_ref.dtype)


def _spatial_attention(x, weight, bias):
    B, C, H, W = x.shape
    kk = weight.shape[2]
    p = kk // 2
    assert H % 2 == 0 and W == 64 and C % 8 == 0
    hpair = H // 2
    w2 = 2 * W

    x_flat = x.reshape(B, C, H * W)

    scale = jnp.array([1.0, 1.0 / C], jnp.float32)
    w_all = weight[0].astype(jnp.float32) * scale[:, None, None]
    mats = _build_conv_mats(w_all, W, kk)

    hh = np.arange(H)[:, None] + np.arange(kk)[None, :] - p
    um = ((hh >= 0) & (hh < H)).astype(np.float32)
    wwv = np.arange(W)[:, None] + np.arange(kk)[None, :] - p
    vm = ((wwv >= 0) & (wwv < W)).astype(np.float32)
    corr = jnp.einsum('hk,ikl,wl->ihw', um, w_all, vm).reshape(2, hpair, w2)

    bias_s = bias.reshape(-1).astype(jnp.float32)

    pad_rows = _round_up(hpair + 4, 8)
    body = functools.partial(_sa_body, C=C, HPAIR=hpair, W2=w2)

    cost = pl.CostEstimate(
        flops=int(B * H * W * (2 * C + 4 * kk * kk + 4)),
        transcendentals=int(B * H * W),
        bytes_accessed=int(B * (C + 1) * H * W * 4 + mats.size * 4),
    )

    out = pl.pallas_call(
        body,
        out_shape=jax.ShapeDtypeStruct((B, 1, hpair, w2), x.dtype),
        grid=(B,),
        in_specs=[
            pl.BlockSpec((1, C, H * W), lambda b: (b, 0, 0)),
            pl.BlockSpec((5, 4 * W, w2), lambda b: (0, 0, 0)),
            pl.BlockSpec((2, hpair, w2), lambda b: (0, 0, 0)),
            pl.BlockSpec(memory_space=pltpu.MemorySpace.SMEM),
        ],
        out_specs=pl.BlockSpec((1, 1, hpair, w2), lambda b: (b, 0, 0, 0)),
        scratch_shapes=[
            pltpu.VMEM((pad_rows, 2 * w2), jnp.float32),
        ],
        compiler_params=pltpu.CompilerParams(
            dimension_semantics=("parallel",),
            vmem_limit_bytes=32 * 1024 * 1024),
        cost_estimate=cost,
    )(x_flat, mats, corr, bias_s)

    return out.reshape(B, 1, H, W)


def kernel(x, weight, bias):
    return _spatial_attention(x, weight, bias)
